# baseline (speedup 1.0000x reference)
import functools

import jax
import jax.numpy as jnp
from jax import lax
from jax.experimental import pallas as pl
from jax.experimental.pallas import tpu as pltpu
from jax.experimental.pallas import tpu_sc as plsc

N_C, N_O, N_E = 10000, 50000, 50000
E = 160000
EP = 163840
D = 128

_NSC = 2
_NT = 16
_B = 128
_EB = E // _B
_EB_SC = _EB // _NSC

def _mesh():
    return plsc.VectorSubcoreMesh(core_axis_name="c", subcore_axis_name="s")


def _sc_params():
    return pltpu.CompilerParams(use_tc_tiling_on_sc=False,
                                needs_layout_passes=False)


def _hist_pad(n):
    rt = 632 * -(-n // (_NT * 632))
    return _NT * rt


def _sc_histograms(idx8, sizes, ones_pat, zeros_pat):
    pads = [_hist_pad(n) for n in sizes]
    outs = [jax.ShapeDtypeStruct((2, np_, 16), jnp.float32) for np_ in pads]

    @functools.partial(
        pl.kernel, out_type=outs, mesh=_mesh(), compiler_params=_sc_params(),
        scratch_types=[
            pltpu.VMEM_SHARED((max(pads), 16), jnp.float32),
            pltpu.VMEM((_B, 16), jnp.float32),
            pltpu.VMEM((8, 128), jnp.int32),
            pltpu.VMEM((8, 128), jnp.int32),
            pltpu.SemaphoreType.DMA,
            pltpu.SemaphoreType.DMA,
            pltpu.SemaphoreType.DMA,
            pltpu.SemaphoreType.DMA,
        ])
    def k(*refs):
        ins = refs[:8]
        ones_h, zeros_h = refs[8], refs[9]
        out_refs = refs[10:18]
        hsp, obuf, ib0, ib1, ds0, ds1, as0, as1 = refs[18:]
        ibs, dss, ass = (ib0, ib1), (ds0, ds1), (as0, as1)
        c = lax.axis_index("c")
        s = lax.axis_index("s")
        pltpu.sync_copy(ones_h, obuf)
        for h in range(8):
            rows_t = pads[h] // _NT
            nch = rows_t // 632

            @pl.loop(0, nch)
            def _zero(kk, s=s, rows_t=rows_t):
                pltpu.sync_copy(zeros_h,
                                hsp.at[pl.ds(s * rows_t + kk * 632, 632)])

            plsc.subcore_barrier()

            def bid(kk, h=h, c=c, s=s):
                return (c * 80 + s + _NT * kk) * 8

            dma = pltpu.async_copy(ins[h].at[pl.ds(bid(0), 8)], ib0, ds0)
            adds_prev = None
            for kk in range(5):
                t = kk & 1
                dma.wait()
                adds = [pltpu.async_copy(obuf, hsp.at[ibs[t].at[j]],
                                         ass[t], add=True)
                        for j in range(8)]
                if adds_prev is not None:
                    for a in adds_prev:
                        a.wait()
                if kk < 4:
                    dma = pltpu.async_copy(ins[h].at[pl.ds(bid(kk + 1), 8)],
                                           ibs[1 - t], dss[1 - t])
                adds_prev = adds
            for a in adds_prev:
                a.wait()

            plsc.subcore_barrier()

            @pl.loop(0, nch)
            def _wb(kk, h=h, c=c, s=s, rows_t=rows_t):
                r0 = s * rows_t + kk * 632
                pltpu.sync_copy(hsp.at[pl.ds(r0, 632)],
                                out_refs[h].at[c, pl.ds(r0, 632)])

            plsc.subcore_barrier()

    return list(k(*idx8, ones_pat, zeros_pat))


def _sc_scatter_rows(xn, src, dst, n_dst, passes, r_rows, zeros_rows):
    npad = _NSC * passes * r_rows
    zsl = (r_rows + 128) // _NT
    wb = r_rows // _NT

    @functools.partial(
        pl.kernel,
        out_type=jax.ShapeDtypeStruct((npad, D), jnp.bfloat16),
        mesh=_mesh(), compiler_params=_sc_params(),
        scratch_types=[
            pltpu.VMEM_SHARED((r_rows + 128, D), jnp.bfloat16),
            pltpu.VMEM((_B, D), jnp.bfloat16),
            pltpu.VMEM((8, 128), jnp.int32),
            pltpu.VMEM((8, 128), jnp.int32),
            pltpu.VMEM((2 * _B + 16,), jnp.int32),
            pltpu.VMEM((2 * _B + 16,), jnp.int32),
            pltpu.VMEM((_B,), jnp.int32),
            pltpu.VMEM((_B,), jnp.int32),
            pltpu.SMEM((2,), jnp.int32),
            pltpu.SemaphoreType.DMA,
        ])
    def k(xn_h, src_h, dst_h, zeros_h, out_h, asp, rows, sg, db,
          csrc, cdst, fs0, fd0, st, gs0):
        c = lax.axis_index("c")
        s = lax.axis_index("s")
        dum = jnp.int32(r_rows) + lax.iota(jnp.int32, 16)

        def drain():
            @pl.when(st[1] == 1)
            def _():
                pltpu.make_async_copy(xn_h.at[fs0], rows, gs0).wait()
                pltpu.sync_copy(rows, asp.at[fd0], add=True)
                st[1] = 0

        @pl.loop(0, passes)
        def _pass(p, c=c, s=s):
            lo = (c * passes + p) * r_rows

            pltpu.sync_copy(zeros_h, asp.at[pl.ds(s * zsl, zsl)])
            st[0] = 0
            st[1] = 0
            plsc.subcore_barrier()

            @pl.loop(0, 10)
            def _scan(kk, lo=lo, s=s):
                b = s + _NT * kk
                pltpu.sync_copy(src_h.at[pl.ds(b * 8, 8)], sg)
                pltpu.sync_copy(dst_h.at[pl.ds(b * 8, 8)], db)
                lane = lax.iota(jnp.int32, 16)
                for gr in range(8):
                    for g in range(8):
                        sv = sg[gr, pl.ds(g * 16, 16)]
                        dv = db[gr, pl.ds(g * 16, 16)]
                        inr = (dv >= lo) & (dv < lo + r_rows)
                        iv = jnp.where(inr, 1, 0)
                        ps = plsc.cumsum(iv)
                        c0 = st[0]
                        pos = jnp.where(inr, c0 + ps - 1, 2 * _B + lane)
                        plsc.store_scatter(csrc, [pos], sv)
                        plsc.store_scatter(cdst, [pos], dv - lo)
                        st[0] = c0 + jnp.sum(iv)

                    @pl.when(st[0] >= _B)
                    def _fire():
                        drain()
                        for g in range(_B // 16):
                            sl = pl.ds(g * 16, 16)
                            fs0[sl] = csrc[sl]
                            fd0[sl] = cdst[sl]
                        pltpu.async_copy(xn_h.at[fs0], rows, gs0)
                        st[1] = 1
                        for g in range(_B // 16):
                            sl = pl.ds(g * 16, 16)
                            sh = pl.ds(_B + g * 16, 16)
                            csrc[sl] = csrc[sh]
                            cdst[sl] = cdst[sh]
                        st[0] = st[0] - _B

            drain()
            r = st[0]
            for g in range(_B // 16):
                sl = pl.ds(g * 16, 16)
                pos = jnp.int32(g * 16) + lax.iota(jnp.int32, 16)
                sel = pos < r
                fs0[sl] = jnp.where(sel, csrc[sl], 0)
                fd0[sl] = jnp.where(sel, cdst[sl], dum)
            pltpu.sync_copy(xn_h.at[fs0], rows)
            pltpu.sync_copy(rows, asp.at[fd0], add=True)
            plsc.subcore_barrier()

            q = c * passes + p
            pltpu.sync_copy(asp.at[pl.ds(s * wb, wb)],
                            out_h.at[pl.ds(q * r_rows + s * wb, wb)])
            plsc.subcore_barrier()

    return k(xn, src, dst, zeros_rows)


def _xnorm_body(x_ref, d0_ref, d1_ref, o_ref):
    deg = d0_ref[:, 0:1] + d1_ref[:, 0:1]
    rs = lax.rsqrt(jnp.maximum(deg, 1.0))
    o_ref[...] = (x_ref[...] * rs).astype(jnp.bfloat16)


def _xnorm(x, dpair):
    n = x.shape[0]
    blk = 1000
    return pl.pallas_call(
        _xnorm_body,
        grid=(n // blk,),
        in_specs=[
            pl.BlockSpec((blk, D), lambda i: (i, 0)),
            pl.BlockSpec((blk, 16), lambda i: (i, 0)),
            pl.BlockSpec((blk, 16), lambda i: (i, 0)),
        ],
        out_specs=pl.BlockSpec((blk, D), lambda i: (i, 0)),
        out_shape=jax.ShapeDtypeStruct((n + 64, D), jnp.bfloat16),
    )(x, dpair[0], dpair[1])


def _dot(a, b):
    return lax.dot_general(a, b, (((1,), (0,)), ((), ())),
                           precision=lax.Precision.HIGHEST,
                           preferred_element_type=jnp.float32)


def _epilogue_body(n_agg, *refs):
    i = 0
    aggs = []
    for _ in range(n_agg):
        agg, d0, d1 = refs[i], refs[i + 1], refs[i + 2]
        i += 3
        deg = d0[:, 0:1] + d1[:, 0:1]
        rs = lax.rsqrt(jnp.maximum(deg, 1.0))
        aggs.append(agg[...].astype(jnp.float32) * rs)
    p1, p2 = refs[i][...], refs[i + 1][...]
    i += 2
    Ws = refs[i:i + n_agg]
    i += n_agg
    bm, G1, G2, G3, bg, WihT, bih, WhhT, bhh = refs[i:i + 9]
    out_h, out_g = refs[i + 9], refs[i + 10]

    m = _dot(aggs[0], Ws[0][...])
    for a in range(1, n_agg):
        m = m + _dot(aggs[a], Ws[a][...])
    m = m + bm[...]

    gate = jax.nn.sigmoid(_dot(m, G1[...]) + _dot(p1, G2[...])
                          + _dot(p2, G3[...]) + bg[...])
    fused = gate * p1 + (1.0 - gate) * p2

    gi = _dot(m, WihT[...]) + bih[...]
    gh = _dot(fused, WhhT[...]) + bhh[...]
    r = jax.nn.sigmoid(gi[:, 0:D] + gh[:, 0:D])
    z = jax.nn.sigmoid(gi[:, D:2 * D] + gh[:, D:2 * D])
    nn = jnp.tanh(gi[:, 2 * D:3 * D] + r * gh[:, 2 * D:3 * D])
    out_h[...] = jnp.maximum((1.0 - z) * nn + z * fused, 0.0)
    out_g[...] = gate


def _epilogue(aggs_dpairs, p1, p2, Ws, bm, G1, G2, G3, bg, WihT, bih,
              WhhT, bhh):
    n = p1.shape[0]
    n_agg = len(aggs_dpairs)
    blk = 1000
    row_spec = lambda w: pl.BlockSpec((blk, w), lambda i: (i, 0))
    full = lambda a, b: pl.BlockSpec((a, b), lambda i: (0, 0))
    in_specs, args = [], []
    for agg, dpair in aggs_dpairs:
        in_specs += [row_spec(D), row_spec(16), row_spec(16)]
        args += [agg, dpair[0], dpair[1]]
    in_specs += [row_spec(D), row_spec(D)]
    args += [p1, p2]
    for W in Ws:
        in_specs.append(full(D, D))
        args.append(W)
    for w_arr, wd in ((bm, D), (G1, D), (G2, D), (G3, D), (bg, D),
                      (WihT, 3 * D), (bih, 3 * D), (WhhT, 3 * D),
                      (bhh, 3 * D)):
        in_specs.append(full(w_arr.shape[0], wd))
        args.append(w_arr)
    return pl.pallas_call(
        functools.partial(_epilogue_body, n_agg),
        grid=(n // blk,),
        in_specs=in_specs,
        out_specs=(row_spec(D), row_spec(D)),
        out_shape=(jax.ShapeDtypeStruct((n, D), jnp.float32),
                   jax.ShapeDtypeStruct((n, D), jnp.float32)),
    )(*args)


def kernel(h_course, h_object, h_enroll, p1_course, p1_object, p1_enroll,
           p2_course, p2_object, p2_enroll,
           src_co, dst_co, src_oc, dst_oc, src_oe, dst_oe, src_eo, dst_eo,
           W_co, b_co, W_oc, b_oc, W_oe, b_oe, W_eo, b_eo,
           W_ih, W_hh, b_ih, b_hh, W_gate, b_gate):
    f32 = jnp.float32
    ones_pat = jnp.zeros((_B, 16), f32).at[:, 0].set(1.0)
    zeros_small = jnp.zeros((632, 16), f32)

    pad_off = jnp.arange(EP - E, dtype=jnp.int32) % 64

    def _pad_idx(a, n):
        return jnp.concatenate([a, n + pad_off]).reshape(EP // 128, 128)

    src_co_p, dst_co_p = _pad_idx(src_co, N_C), _pad_idx(dst_co, N_O)
    src_eo_p, dst_eo_p = _pad_idx(src_eo, N_E), _pad_idx(dst_eo, N_O)
    src_oc_p, dst_oc_p = _pad_idx(src_oc, N_O), _pad_idx(dst_oc, N_C)
    src_oe_p, dst_oe_p = _pad_idx(src_oe, N_O), _pad_idx(dst_oe, N_E)

    idx8 = [src_co_p, dst_co_p, src_eo_p, dst_eo_p,
            src_oc_p, dst_oc_p, src_oe_p, dst_oe_p]
    sizes = [N_C, N_O, N_E, N_O, N_O, N_C, N_O, N_E]
    hists = _sc_histograms(idx8, sizes, ones_pat, zeros_small)
    (d_s_co, d_d_co, d_s_eo, d_d_eo,
     d_s_oc, d_d_oc, d_s_oe, d_d_oe) = hists

    xn_co = _xnorm(h_course, d_s_co)
    xn_eo = _xnorm(h_enroll, d_s_eo)
    xn_oc = _xnorm(h_object, d_s_oc)
    xn_oe = _xnorm(h_object, d_s_oe)

    r_big, p_big = 25088, 1
    r_sml, p_sml = 5120, 1
    zeros_big = jnp.zeros(((r_big + 128) // _NT, D), jnp.bfloat16)
    zeros_sml = jnp.zeros(((r_sml + 128) // _NT, D), jnp.bfloat16)
    agg_co = _sc_scatter_rows(xn_co, src_co_p, dst_co_p, N_O, p_big, r_big,
                              zeros_big)
    agg_eo = _sc_scatter_rows(xn_eo, src_eo_p, dst_eo_p, N_O, p_big, r_big,
                              zeros_big)
    agg_oc = _sc_scatter_rows(xn_oc, src_oc_p, dst_oc_p, N_C, p_sml, r_sml,
                              zeros_sml)
    agg_oe = _sc_scatter_rows(xn_oe, src_oe_p, dst_oe_p, N_E, p_big, r_big,
                              zeros_big)

    WgT = W_gate.T
    G1, G2, G3 = WgT[0:D], WgT[D:2 * D], WgT[2 * D:3 * D]
    WihT, WhhT = W_ih.T, W_hh.T
    bg = b_gate.reshape(1, D)
    bih = b_ih.reshape(1, 3 * D)
    bhh = b_hh.reshape(1, 3 * D)

    h_o, g_o = _epilogue([(agg_co, d_d_co), (agg_eo, d_d_eo)],
                         p1_object, p2_object, [W_co, W_eo],
                         (b_co + b_eo).reshape(1, D),
                         G1, G2, G3, bg, WihT, bih, WhhT, bhh)
    h_c, g_c = _epilogue([(agg_oc, d_d_oc)],
                         p1_course, p2_course, [W_oc],
                         b_oc.reshape(1, D),
                         G1, G2, G3, bg, WihT, bih, WhhT, bhh)
    h_e, g_e = _epilogue([(agg_oe, d_d_oe)],
                         p1_enroll, p2_enroll, [W_oe],
                         b_oe.reshape(1, D),
                         G1, G2, G3, bg, WihT, bih, WhhT, bhh)
    return (h_c, h_o, h_e, g_c, g_o, g_e)


# speedup vs baseline: 1.3797x; 1.3797x over previous
import functools

import jax
import jax.numpy as jnp
from jax import lax
from jax.experimental import pallas as pl
from jax.experimental.pallas import tpu as pltpu
from jax.experimental.pallas import tpu_sc as plsc

N_C, N_O, N_E = 10000, 50000, 50000
E = 160000
EP = 163840
D = 128

_NSC = 2
_NT = 16
_B = 128
_EB = E // _B
_EB_SC = _EB // _NSC

def _mesh():
    return plsc.VectorSubcoreMesh(core_axis_name="c", subcore_axis_name="s")


def _sc_params():
    return pltpu.CompilerParams(use_tc_tiling_on_sc=False,
                                needs_layout_passes=False)


def _hist_pad(n):
    rt = 632 * -(-n // (_NT * 632))
    return _NT * rt


def _sc_histograms(idx8, sizes, ones_pat, zeros_pat):
    pads = [_hist_pad(n) for n in sizes]
    outs = [jax.ShapeDtypeStruct((2, np_, 16), jnp.float32) for np_ in pads]

    @functools.partial(
        pl.kernel, out_type=outs, mesh=_mesh(), compiler_params=_sc_params(),
        scratch_types=[
            pltpu.VMEM_SHARED((max(pads), 16), jnp.float32),
            pltpu.VMEM((_B, 16), jnp.float32),
            pltpu.VMEM((8, 128), jnp.int32),
            pltpu.VMEM((8, 128), jnp.int32),
            pltpu.SemaphoreType.DMA,
            pltpu.SemaphoreType.DMA,
            pltpu.SemaphoreType.DMA,
            pltpu.SemaphoreType.DMA,
        ])
    def k(*refs):
        ins = refs[:8]
        ones_h, zeros_h = refs[8], refs[9]
        out_refs = refs[10:18]
        hsp, obuf, ib0, ib1, ds0, ds1, as0, as1 = refs[18:]
        ibs, dss, ass = (ib0, ib1), (ds0, ds1), (as0, as1)
        c = lax.axis_index("c")
        s = lax.axis_index("s")
        pltpu.sync_copy(ones_h, obuf)
        for h in range(8):
            rows_t = pads[h] // _NT
            nch = rows_t // 632

            @pl.loop(0, nch)
            def _zero(kk, s=s, rows_t=rows_t):
                pltpu.sync_copy(zeros_h,
                                hsp.at[pl.ds(s * rows_t + kk * 632, 632)])

            plsc.subcore_barrier()

            def bid(kk, h=h, c=c, s=s):
                return (c * 80 + s + _NT * kk) * 8

            dma = pltpu.async_copy(ins[h].at[pl.ds(bid(0), 8)], ib0, ds0)
            adds_prev = None
            for kk in range(5):
                t = kk & 1
                dma.wait()
                adds = [pltpu.async_copy(obuf, hsp.at[ibs[t].at[j]],
                                         ass[t], add=True)
                        for j in range(8)]
                if adds_prev is not None:
                    for a in adds_prev:
                        a.wait()
                if kk < 4:
                    dma = pltpu.async_copy(ins[h].at[pl.ds(bid(kk + 1), 8)],
                                           ibs[1 - t], dss[1 - t])
                adds_prev = adds
            for a in adds_prev:
                a.wait()

            plsc.subcore_barrier()

            @pl.loop(0, nch)
            def _wb(kk, h=h, c=c, s=s, rows_t=rows_t):
                r0 = s * rows_t + kk * 632
                pltpu.sync_copy(hsp.at[pl.ds(r0, 632)],
                                out_refs[h].at[c, pl.ds(r0, 632)])

            plsc.subcore_barrier()

    return list(k(*idx8, ones_pat, zeros_pat))


def _sc_scatter_rows(xn, src, dst, n_dst, passes, r_rows, zeros_rows):
    npad = _NSC * passes * r_rows
    zsl = (r_rows + 128) // _NT
    wb = r_rows // _NT

    @functools.partial(
        pl.kernel,
        out_type=jax.ShapeDtypeStruct((npad, D), jnp.bfloat16),
        mesh=_mesh(), compiler_params=_sc_params(),
        scratch_types=[
            pltpu.VMEM_SHARED((r_rows + 128, D), jnp.bfloat16),
            pltpu.VMEM((_B, D), jnp.bfloat16),
            pltpu.VMEM((8, 128), jnp.int32),
            pltpu.VMEM((8, 128), jnp.int32),
            pltpu.VMEM((2 * _B + 16,), jnp.int32),
            pltpu.VMEM((2 * _B + 16,), jnp.int32),
            pltpu.VMEM((_B,), jnp.int32),
            pltpu.VMEM((_B,), jnp.int32),
            pltpu.SMEM((2,), jnp.int32),
            pltpu.SemaphoreType.DMA,
        ])
    def k(xn_h, src_h, dst_h, zeros_h, out_h, asp, rows, sg, db,
          csrc, cdst, fs0, fd0, st, gs0):
        c = lax.axis_index("c")
        s = lax.axis_index("s")
        dum = jnp.int32(r_rows) + lax.iota(jnp.int32, 16)

        def drain():
            @pl.when(st[1] == 1)
            def _():
                pltpu.make_async_copy(xn_h.at[fs0], rows, gs0).wait()
                pltpu.sync_copy(rows, asp.at[fd0], add=True)
                st[1] = 0

        @pl.loop(0, passes)
        def _pass(p, c=c, s=s):
            lo = (c * passes + p) * r_rows

            pltpu.sync_copy(zeros_h, asp.at[pl.ds(s * zsl, zsl)])
            st[0] = 0
            st[1] = 0
            plsc.subcore_barrier()

            @pl.loop(0, 10)
            def _scan(kk, lo=lo, s=s):
                b = s + _NT * kk
                pltpu.sync_copy(src_h.at[pl.ds(b * 8, 8)], sg)
                pltpu.sync_copy(dst_h.at[pl.ds(b * 8, 8)], db)
                lane = lax.iota(jnp.int32, 16)
                for gr in range(8):
                    for g in range(8):
                        sv = sg[gr, pl.ds(g * 16, 16)]
                        dv = db[gr, pl.ds(g * 16, 16)]
                        inr = (dv >= lo) & (dv < lo + r_rows)
                        iv = jnp.where(inr, 1, 0)
                        ps = plsc.cumsum(iv)
                        c0 = st[0]
                        pos = jnp.where(inr, c0 + ps - 1, 2 * _B + lane)
                        plsc.store_scatter(csrc, [pos], sv)
                        plsc.store_scatter(cdst, [pos], dv - lo)
                        st[0] = c0 + jnp.sum(iv)

                    @pl.when(st[0] >= _B)
                    def _fire():
                        drain()
                        for g in range(_B // 16):
                            sl = pl.ds(g * 16, 16)
                            fs0[sl] = csrc[sl]
                            fd0[sl] = cdst[sl]
                        pltpu.async_copy(xn_h.at[fs0], rows, gs0)
                        st[1] = 1
                        for g in range(_B // 16):
                            sl = pl.ds(g * 16, 16)
                            sh = pl.ds(_B + g * 16, 16)
                            csrc[sl] = csrc[sh]
                            cdst[sl] = cdst[sh]
                        st[0] = st[0] - _B

            drain()
            r = st[0]
            for g in range(_B // 16):
                sl = pl.ds(g * 16, 16)
                pos = jnp.int32(g * 16) + lax.iota(jnp.int32, 16)
                sel = pos < r
                fs0[sl] = jnp.where(sel, csrc[sl], 0)
                fd0[sl] = jnp.where(sel, cdst[sl], dum)
            pltpu.sync_copy(xn_h.at[fs0], rows)
            pltpu.sync_copy(rows, asp.at[fd0], add=True)
            plsc.subcore_barrier()

            q = c * passes + p
            pltpu.sync_copy(asp.at[pl.ds(s * wb, wb)],
                            out_h.at[pl.ds(q * r_rows + s * wb, wb)])
            plsc.subcore_barrier()

    return k(xn, src, dst, zeros_rows)


def _xnorm_body(x_ref, d0_ref, d1_ref, o_ref):
    deg = d0_ref[:, 0:1] + d1_ref[:, 0:1]
    rs = lax.rsqrt(jnp.maximum(deg, 1.0))
    o_ref[...] = (x_ref[...] * rs).astype(jnp.bfloat16)


def _xnorm(x, dpair):
    n = x.shape[0]
    blk = 1000
    return pl.pallas_call(
        _xnorm_body,
        grid=(n // blk,),
        in_specs=[
            pl.BlockSpec((blk, D), lambda i: (i, 0)),
            pl.BlockSpec((blk, 16), lambda i: (i, 0)),
            pl.BlockSpec((blk, 16), lambda i: (i, 0)),
        ],
        out_specs=pl.BlockSpec((blk, D), lambda i: (i, 0)),
        out_shape=jax.ShapeDtypeStruct((n + 64, D), jnp.bfloat16),
    )(x, dpair[0], dpair[1])


def _dot(a, b):
    return lax.dot_general(a.astype(jnp.bfloat16), b.astype(jnp.bfloat16),
                           (((1,), (0,)), ((), ())),
                           preferred_element_type=jnp.float32)


def _epilogue_body(n_agg, *refs):
    i = 0
    aggs = []
    for _ in range(n_agg):
        agg, d0, d1 = refs[i], refs[i + 1], refs[i + 2]
        i += 3
        deg = d0[:, 0:1] + d1[:, 0:1]
        rs = lax.rsqrt(jnp.maximum(deg, 1.0))
        aggs.append(agg[...].astype(jnp.float32) * rs)
    p1, p2 = refs[i][...], refs[i + 1][...]
    i += 2
    Ws = refs[i:i + n_agg]
    i += n_agg
    bm, G1, G2, G3, bg, WihT, bih, WhhT, bhh = refs[i:i + 9]
    out_h, out_g = refs[i + 9], refs[i + 10]

    m = _dot(aggs[0], Ws[0][...])
    for a in range(1, n_agg):
        m = m + _dot(aggs[a], Ws[a][...])
    m = m + bm[...]

    gate = jax.nn.sigmoid(_dot(m, G1[...]) + _dot(p1, G2[...])
                          + _dot(p2, G3[...]) + bg[...])
    fused = gate * p1 + (1.0 - gate) * p2

    gi = _dot(m, WihT[...]) + bih[...]
    gh = _dot(fused, WhhT[...]) + bhh[...]
    r = jax.nn.sigmoid(gi[:, 0:D] + gh[:, 0:D])
    z = jax.nn.sigmoid(gi[:, D:2 * D] + gh[:, D:2 * D])
    nn = jnp.tanh(gi[:, 2 * D:3 * D] + r * gh[:, 2 * D:3 * D])
    out_h[...] = jnp.maximum((1.0 - z) * nn + z * fused, 0.0)
    out_g[...] = gate


def _epilogue(aggs_dpairs, p1, p2, Ws, bm, G1, G2, G3, bg, WihT, bih,
              WhhT, bhh):
    n = p1.shape[0]
    n_agg = len(aggs_dpairs)
    blk = 1000
    row_spec = lambda w: pl.BlockSpec((blk, w), lambda i: (i, 0))
    full = lambda a, b: pl.BlockSpec((a, b), lambda i: (0, 0))
    in_specs, args = [], []
    for agg, dpair in aggs_dpairs:
        in_specs += [row_spec(D), row_spec(16), row_spec(16)]
        args += [agg, dpair[0], dpair[1]]
    in_specs += [row_spec(D), row_spec(D)]
    args += [p1, p2]
    for W in Ws:
        in_specs.append(full(D, D))
        args.append(W)
    for w_arr, wd in ((bm, D), (G1, D), (G2, D), (G3, D), (bg, D),
                      (WihT, 3 * D), (bih, 3 * D), (WhhT, 3 * D),
                      (bhh, 3 * D)):
        in_specs.append(full(w_arr.shape[0], wd))
        args.append(w_arr)
    return pl.pallas_call(
        functools.partial(_epilogue_body, n_agg),
        grid=(n // blk,),
        in_specs=in_specs,
        out_specs=(row_spec(D), row_spec(D)),
        out_shape=(jax.ShapeDtypeStruct((n, D), jnp.float32),
                   jax.ShapeDtypeStruct((n, D), jnp.float32)),
    )(*args)


def kernel(h_course, h_object, h_enroll, p1_course, p1_object, p1_enroll,
           p2_course, p2_object, p2_enroll,
           src_co, dst_co, src_oc, dst_oc, src_oe, dst_oe, src_eo, dst_eo,
           W_co, b_co, W_oc, b_oc, W_oe, b_oe, W_eo, b_eo,
           W_ih, W_hh, b_ih, b_hh, W_gate, b_gate):
    f32 = jnp.float32
    ones_pat = jnp.zeros((_B, 16), f32).at[:, 0].set(1.0)
    zeros_small = jnp.zeros((632, 16), f32)

    pad_off = jnp.arange(EP - E, dtype=jnp.int32) % 64

    def _pad_idx(a, n):
        return jnp.concatenate([a, n + pad_off]).reshape(EP // 128, 128)

    src_co_p, dst_co_p = _pad_idx(src_co, N_C), _pad_idx(dst_co, N_O)
    src_eo_p, dst_eo_p = _pad_idx(src_eo, N_E), _pad_idx(dst_eo, N_O)
    src_oc_p, dst_oc_p = _pad_idx(src_oc, N_O), _pad_idx(dst_oc, N_C)
    src_oe_p, dst_oe_p = _pad_idx(src_oe, N_O), _pad_idx(dst_oe, N_E)

    idx8 = [src_co_p, dst_co_p, src_eo_p, dst_eo_p,
            src_oc_p, dst_oc_p, src_oe_p, dst_oe_p]
    sizes = [N_C, N_O, N_E, N_O, N_O, N_C, N_O, N_E]
    hists = _sc_histograms(idx8, sizes, ones_pat, zeros_small)
    (d_s_co, d_d_co, d_s_eo, d_d_eo,
     d_s_oc, d_d_oc, d_s_oe, d_d_oe) = hists

    xn_co = _xnorm(h_course, d_s_co)
    xn_eo = _xnorm(h_enroll, d_s_eo)
    xn_oc = _xnorm(h_object, d_s_oc)
    xn_oe = _xnorm(h_object, d_s_oe)

    r_big, p_big = 25088, 1
    r_sml, p_sml = 5120, 1
    zeros_big = jnp.zeros(((r_big + 128) // _NT, D), jnp.bfloat16)
    zeros_sml = jnp.zeros(((r_sml + 128) // _NT, D), jnp.bfloat16)
    agg_co = _sc_scatter_rows(xn_co, src_co_p, dst_co_p, N_O, p_big, r_big,
                              zeros_big)
    agg_eo = _sc_scatter_rows(xn_eo, src_eo_p, dst_eo_p, N_O, p_big, r_big,
                              zeros_big)
    agg_oc = _sc_scatter_rows(xn_oc, src_oc_p, dst_oc_p, N_C, p_sml, r_sml,
                              zeros_sml)
    agg_oe = _sc_scatter_rows(xn_oe, src_oe_p, dst_oe_p, N_E, p_big, r_big,
                              zeros_big)

    WgT = W_gate.T
    G1, G2, G3 = WgT[0:D], WgT[D:2 * D], WgT[2 * D:3 * D]
    WihT, WhhT = W_ih.T, W_hh.T
    bg = b_gate.reshape(1, D)
    bih = b_ih.reshape(1, 3 * D)
    bhh = b_hh.reshape(1, 3 * D)

    h_o, g_o = _epilogue([(agg_co, d_d_co), (agg_eo, d_d_eo)],
                         p1_object, p2_object, [W_co, W_eo],
                         (b_co + b_eo).reshape(1, D),
                         G1, G2, G3, bg, WihT, bih, WhhT, bhh)
    h_c, g_c = _epilogue([(agg_oc, d_d_oc)],
                         p1_course, p2_course, [W_oc],
                         b_oc.reshape(1, D),
                         G1, G2, G3, bg, WihT, bih, WhhT, bhh)
    h_e, g_e = _epilogue([(agg_oe, d_d_oe)],
                         p1_enroll, p2_enroll, [W_oe],
                         b_oe.reshape(1, D),
                         G1, G2, G3, bg, WihT, bih, WhhT, bhh)
    return (h_c, h_o, h_e, g_c, g_o, g_e)


# speedup vs baseline: 1.5999x; 1.1596x over previous
import functools

import jax
import jax.numpy as jnp
from jax import lax
from jax.experimental import pallas as pl
from jax.experimental.pallas import tpu as pltpu
from jax.experimental.pallas import tpu_sc as plsc

N_C, N_O, N_E = 10000, 50000, 50000
E = 160000
EP = 163840
D = 128

_NSC = 2
_NT = 16
_B = 128
_EB = E // _B
_EB_SC = _EB // _NSC

def _mesh():
    return plsc.VectorSubcoreMesh(core_axis_name="c", subcore_axis_name="s")


def _sc_params():
    return pltpu.CompilerParams(use_tc_tiling_on_sc=False,
                                needs_layout_passes=False)


def _hist_pad(n):
    rt = 624 * -(-n // (_NT * 624))
    return _NT * rt


def _sc_histograms(idx8, sizes, ones_pat, zeros_pat):
    pads = [_hist_pad(n) for n in sizes]
    outs = [jax.ShapeDtypeStruct((2, np_), jnp.float32) for np_ in pads]

    @functools.partial(
        pl.kernel, out_type=outs, mesh=_mesh(), compiler_params=_sc_params(),
        scratch_types=[
            pltpu.VMEM_SHARED((max(pads), 16), jnp.float32),
            pltpu.VMEM((_B, 16), jnp.float32),
            pltpu.VMEM((8, 128), jnp.int32),
            pltpu.VMEM((8, 128), jnp.int32),
            pltpu.VMEM((624, 16), jnp.float32),
            pltpu.VMEM((624,), jnp.float32),
            pltpu.SemaphoreType.DMA,
            pltpu.SemaphoreType.DMA,
            pltpu.SemaphoreType.DMA,
            pltpu.SemaphoreType.DMA,
        ])
    def k(*refs):
        ins = refs[:8]
        ones_h, zeros_h = refs[8], refs[9]
        out_refs = refs[10:18]
        hsp, obuf, ib0, ib1, xin, xout, ds0, ds1, as0, as1 = refs[18:]
        ibs, dss, ass = (ib0, ib1), (ds0, ds1), (as0, as1)
        c = lax.axis_index("c")
        s = lax.axis_index("s")
        pltpu.sync_copy(ones_h, obuf)
        zer16 = lax.iota(jnp.int32, 16) * 0
        for h in range(8):
            rows_t = pads[h] // _NT
            nch = rows_t // 624

            @pl.loop(0, nch)
            def _zero(kk, s=s, rows_t=rows_t):
                pltpu.sync_copy(zeros_h,
                                hsp.at[pl.ds(s * rows_t + kk * 624, 624)])

            plsc.subcore_barrier()

            def bid(kk, h=h, c=c, s=s):
                return (c * 80 + s + _NT * kk) * 8

            dma = pltpu.async_copy(ins[h].at[pl.ds(bid(0), 8)], ib0, ds0)
            adds_prev = None
            for kk in range(5):
                t = kk & 1
                dma.wait()
                adds = [pltpu.async_copy(obuf, hsp.at[ibs[t].at[j]],
                                         ass[t], add=True)
                        for j in range(8)]
                if adds_prev is not None:
                    for a in adds_prev:
                        a.wait()
                if kk < 4:
                    dma = pltpu.async_copy(ins[h].at[pl.ds(bid(kk + 1), 8)],
                                           ibs[1 - t], dss[1 - t])
                adds_prev = adds
            for a in adds_prev:
                a.wait()

            plsc.subcore_barrier()

            @pl.loop(0, nch)
            def _wb(kk, h=h, c=c, s=s, rows_t=rows_t):
                r0 = s * rows_t + kk * 624
                pltpu.sync_copy(hsp.at[pl.ds(r0, 624)], xin)
                for g in range(624 // 16):
                    idx0 = jnp.int32(g * 16) + lax.iota(jnp.int32, 16)
                    xout[pl.ds(g * 16, 16)] = plsc.load_gather(
                        xin, [idx0, zer16])
                pltpu.sync_copy(xout, out_refs[h].at[c, pl.ds(r0, 624)])

            plsc.subcore_barrier()

    return list(k(*idx8, ones_pat, zeros_pat))


def _sc_scatter_rows(xn, src, dst, n_dst, passes, r_rows, zeros_rows):
    npad = _NSC * passes * r_rows
    zsl = (r_rows + 128) // _NT
    wb = r_rows // _NT

    @functools.partial(
        pl.kernel,
        out_type=jax.ShapeDtypeStruct((npad, D), jnp.bfloat16),
        mesh=_mesh(), compiler_params=_sc_params(),
        scratch_types=[
            pltpu.VMEM_SHARED((r_rows + 128, D), jnp.bfloat16),
            pltpu.VMEM((_B, D), jnp.bfloat16),
            pltpu.VMEM((8, 128), jnp.int32),
            pltpu.VMEM((8, 128), jnp.int32),
            pltpu.VMEM((2 * _B + 16,), jnp.int32),
            pltpu.VMEM((2 * _B + 16,), jnp.int32),
            pltpu.VMEM((_B,), jnp.int32),
            pltpu.VMEM((_B,), jnp.int32),
            pltpu.SMEM((2,), jnp.int32),
            pltpu.SemaphoreType.DMA,
        ])
    def k(xn_h, src_h, dst_h, zeros_h, out_h, asp, rows, sg, db,
          csrc, cdst, fs0, fd0, st, gs0):
        c = lax.axis_index("c")
        s = lax.axis_index("s")
        dum = jnp.int32(r_rows) + lax.iota(jnp.int32, 16)

        def drain():
            @pl.when(st[1] == 1)
            def _():
                pltpu.make_async_copy(xn_h.at[fs0], rows, gs0).wait()
                pltpu.sync_copy(rows, asp.at[fd0], add=True)
                st[1] = 0

        @pl.loop(0, passes)
        def _pass(p, c=c, s=s):
            lo = (c * passes + p) * r_rows

            pltpu.sync_copy(zeros_h, asp.at[pl.ds(s * zsl, zsl)])
            st[0] = 0
            st[1] = 0
            plsc.subcore_barrier()

            @pl.loop(0, 10)
            def _scan(kk, lo=lo, s=s):
                b = s + _NT * kk
                pltpu.sync_copy(src_h.at[pl.ds(b * 8, 8)], sg)
                pltpu.sync_copy(dst_h.at[pl.ds(b * 8, 8)], db)
                lane = lax.iota(jnp.int32, 16)
                for gr in range(8):
                    for g in range(8):
                        sv = sg[gr, pl.ds(g * 16, 16)]
                        dv = db[gr, pl.ds(g * 16, 16)]
                        inr = (dv >= lo) & (dv < lo + r_rows)
                        iv = jnp.where(inr, 1, 0)
                        ps = plsc.cumsum(iv)
                        c0 = st[0]
                        pos = jnp.where(inr, c0 + ps - 1, 2 * _B + lane)
                        plsc.store_scatter(csrc, [pos], sv)
                        plsc.store_scatter(cdst, [pos], dv - lo)
                        st[0] = c0 + jnp.sum(iv)

                    @pl.when(st[0] >= _B)
                    def _fire():
                        drain()
                        for g in range(_B // 16):
                            sl = pl.ds(g * 16, 16)
                            fs0[sl] = csrc[sl]
                            fd0[sl] = cdst[sl]
                        pltpu.async_copy(xn_h.at[fs0], rows, gs0)
                        st[1] = 1
                        for g in range(_B // 16):
                            sl = pl.ds(g * 16, 16)
                            sh = pl.ds(_B + g * 16, 16)
                            csrc[sl] = csrc[sh]
                            cdst[sl] = cdst[sh]
                        st[0] = st[0] - _B

            drain()
            r = st[0]
            for g in range(_B // 16):
                sl = pl.ds(g * 16, 16)
                pos = jnp.int32(g * 16) + lax.iota(jnp.int32, 16)
                sel = pos < r
                fs0[sl] = jnp.where(sel, csrc[sl], 0)
                fd0[sl] = jnp.where(sel, cdst[sl], dum)
            pltpu.sync_copy(xn_h.at[fs0], rows)
            pltpu.sync_copy(rows, asp.at[fd0], add=True)
            plsc.subcore_barrier()

            q = c * passes + p
            pltpu.sync_copy(asp.at[pl.ds(s * wb, wb)],
                            out_h.at[pl.ds(q * r_rows + s * wb, wb)])
            plsc.subcore_barrier()

    return k(xn, src, dst, zeros_rows)


def _xnorm_body(x_ref, d0_ref, d1_ref, o_ref):
    deg = d0_ref[...] + d1_ref[...]
    rs = lax.rsqrt(jnp.maximum(deg, 1.0))
    o_ref[...] = (x_ref[...] * rs).astype(jnp.bfloat16)


def _xnorm(x, dpair):
    n = x.shape[0]
    blk = 1000
    return pl.pallas_call(
        _xnorm_body,
        grid=(n // blk,),
        in_specs=[
            pl.BlockSpec((blk, D), lambda i: (i, 0)),
            pl.BlockSpec((blk, 1), lambda i: (i, 0)),
            pl.BlockSpec((blk, 1), lambda i: (i, 0)),
        ],
        out_specs=pl.BlockSpec((blk, D), lambda i: (i, 0)),
        out_shape=jax.ShapeDtypeStruct((n + 64, D), jnp.bfloat16),
    )(x, dpair[0], dpair[1])


def _dot(a, b):
    return lax.dot_general(a.astype(jnp.bfloat16), b.astype(jnp.bfloat16),
                           (((1,), (0,)), ((), ())),
                           preferred_element_type=jnp.float32)


def _epilogue_body(n_agg, *refs):
    i = 0
    aggs = []
    for _ in range(n_agg):
        agg, d0, d1 = refs[i], refs[i + 1], refs[i + 2]
        i += 3
        deg = d0[...] + d1[...]
        rs = lax.rsqrt(jnp.maximum(deg, 1.0))
        aggs.append(agg[...].astype(jnp.float32) * rs)
    p1, p2 = refs[i][...], refs[i + 1][...]
    i += 2
    Ws = refs[i:i + n_agg]
    i += n_agg
    bm, G1, G2, G3, bg, WihT, bih, WhhT, bhh = refs[i:i + 9]
    out_h, out_g = refs[i + 9], refs[i + 10]

    m = _dot(aggs[0], Ws[0][...])
    for a in range(1, n_agg):
        m = m + _dot(aggs[a], Ws[a][...])
    m = m + bm[...]

    gate = jax.nn.sigmoid(_dot(m, G1[...]) + _dot(p1, G2[...])
                          + _dot(p2, G3[...]) + bg[...])
    fused = gate * p1 + (1.0 - gate) * p2

    gi = _dot(m, WihT[...]) + bih[...]
    gh = _dot(fused, WhhT[...]) + bhh[...]
    r = jax.nn.sigmoid(gi[:, 0:D] + gh[:, 0:D])
    z = jax.nn.sigmoid(gi[:, D:2 * D] + gh[:, D:2 * D])
    nn = jnp.tanh(gi[:, 2 * D:3 * D] + r * gh[:, 2 * D:3 * D])
    out_h[...] = jnp.maximum((1.0 - z) * nn + z * fused, 0.0)
    out_g[...] = gate


def _epilogue(aggs_dpairs, p1, p2, Ws, bm, G1, G2, G3, bg, WihT, bih,
              WhhT, bhh):
    n = p1.shape[0]
    n_agg = len(aggs_dpairs)
    blk = 1000
    row_spec = lambda w: pl.BlockSpec((blk, w), lambda i: (i, 0))
    full = lambda a, b: pl.BlockSpec((a, b), lambda i: (0, 0))
    in_specs, args = [], []
    for agg, dpair in aggs_dpairs:
        in_specs += [row_spec(D), row_spec(1), row_spec(1)]
        args += [agg, dpair[0], dpair[1]]
    in_specs += [row_spec(D), row_spec(D)]
    args += [p1, p2]
    for W in Ws:
        in_specs.append(full(D, D))
        args.append(W)
    for w_arr, wd in ((bm, D), (G1, D), (G2, D), (G3, D), (bg, D),
                      (WihT, 3 * D), (bih, 3 * D), (WhhT, 3 * D),
                      (bhh, 3 * D)):
        in_specs.append(full(w_arr.shape[0], wd))
        args.append(w_arr)
    return pl.pallas_call(
        functools.partial(_epilogue_body, n_agg),
        grid=(n // blk,),
        in_specs=in_specs,
        out_specs=(row_spec(D), row_spec(D)),
        out_shape=(jax.ShapeDtypeStruct((n, D), jnp.float32),
                   jax.ShapeDtypeStruct((n, D), jnp.float32)),
    )(*args)


def kernel(h_course, h_object, h_enroll, p1_course, p1_object, p1_enroll,
           p2_course, p2_object, p2_enroll,
           src_co, dst_co, src_oc, dst_oc, src_oe, dst_oe, src_eo, dst_eo,
           W_co, b_co, W_oc, b_oc, W_oe, b_oe, W_eo, b_eo,
           W_ih, W_hh, b_ih, b_hh, W_gate, b_gate):
    f32 = jnp.float32
    ones_pat = jnp.zeros((_B, 16), f32).at[:, 0].set(1.0)
    zeros_small = jnp.zeros((624, 16), f32)

    pad_off = jnp.arange(EP - E, dtype=jnp.int32) % 64

    def _pad_idx(a, n):
        return jnp.concatenate([a, n + pad_off]).reshape(EP // 128, 128)

    src_co_p, dst_co_p = _pad_idx(src_co, N_C), _pad_idx(dst_co, N_O)
    src_eo_p, dst_eo_p = _pad_idx(src_eo, N_E), _pad_idx(dst_eo, N_O)
    src_oc_p, dst_oc_p = _pad_idx(src_oc, N_O), _pad_idx(dst_oc, N_C)
    src_oe_p, dst_oe_p = _pad_idx(src_oe, N_O), _pad_idx(dst_oe, N_E)

    idx8 = [src_co_p, dst_co_p, src_eo_p, dst_eo_p,
            src_oc_p, dst_oc_p, src_oe_p, dst_oe_p]
    sizes = [N_C, N_O, N_E, N_O, N_O, N_C, N_O, N_E]
    hists = _sc_histograms(idx8, sizes, ones_pat, zeros_small)
    hists = [(h[0][:, None], h[1][:, None]) for h in hists]
    (d_s_co, d_d_co, d_s_eo, d_d_eo,
     d_s_oc, d_d_oc, d_s_oe, d_d_oe) = hists

    xn_co = _xnorm(h_course, d_s_co)
    xn_eo = _xnorm(h_enroll, d_s_eo)
    xn_oc = _xnorm(h_object, d_s_oc)
    xn_oe = _xnorm(h_object, d_s_oe)

    r_big, p_big = 25088, 1
    r_sml, p_sml = 5120, 1
    zeros_big = jnp.zeros(((r_big + 128) // _NT, D), jnp.bfloat16)
    zeros_sml = jnp.zeros(((r_sml + 128) // _NT, D), jnp.bfloat16)
    agg_co = _sc_scatter_rows(xn_co, src_co_p, dst_co_p, N_O, p_big, r_big,
                              zeros_big)
    agg_eo = _sc_scatter_rows(xn_eo, src_eo_p, dst_eo_p, N_O, p_big, r_big,
                              zeros_big)
    agg_oc = _sc_scatter_rows(xn_oc, src_oc_p, dst_oc_p, N_C, p_sml, r_sml,
                              zeros_sml)
    agg_oe = _sc_scatter_rows(xn_oe, src_oe_p, dst_oe_p, N_E, p_big, r_big,
                              zeros_big)

    WgT = W_gate.T
    G1, G2, G3 = WgT[0:D], WgT[D:2 * D], WgT[2 * D:3 * D]
    WihT, WhhT = W_ih.T, W_hh.T
    bg = b_gate.reshape(1, D)
    bih = b_ih.reshape(1, 3 * D)
    bhh = b_hh.reshape(1, 3 * D)

    h_o, g_o = _epilogue([(agg_co, d_d_co), (agg_eo, d_d_eo)],
                         p1_object, p2_object, [W_co, W_eo],
                         (b_co + b_eo).reshape(1, D),
                         G1, G2, G3, bg, WihT, bih, WhhT, bhh)
    h_c, g_c = _epilogue([(agg_oc, d_d_oc)],
                         p1_course, p2_course, [W_oc],
                         b_oc.reshape(1, D),
                         G1, G2, G3, bg, WihT, bih, WhhT, bhh)
    h_e, g_e = _epilogue([(agg_oe, d_d_oe)],
                         p1_enroll, p2_enroll, [W_oe],
                         b_oe.reshape(1, D),
                         G1, G2, G3, bg, WihT, bih, WhhT, bhh)
    return (h_c, h_o, h_e, g_c, g_o, g_e)
